# baseline (speedup 1.0000x reference)
import functools

import jax
import jax.numpy as jnp
from jax import lax
from jax.experimental import pallas as pl
from jax.experimental.pallas import tpu as pltpu
from jax.experimental.pallas import tpu_sc as plsc

N = 100000
E = 1600000
D_IN = 128
D_H = 512
D_FF = 2048
D_OUT = 2000

NC, NS = 2, 16
NW = NC * NS
EPW = E // NW

NPAD = 100352
LOC_BITS = 13
LOC_MASK = (1 << LOC_BITS) - 1
EP = E + 256

K1_BLK = 2000
K2_BLK = 2048
CNT_COL = 512

_mesh = plsc.VectorSubcoreMesh(core_axis_name="c", subcore_axis_name="s")
_sc_params = pltpu.CompilerParams(needs_layout_passes=False)


_GDN = lax.GatherDimensionNumbers(
    offset_dims=(), collapsed_slice_dims=(0,), start_index_map=(0,))


def _take16(v, idx):
  return lax.gather(v, idx[:, None], _GDN, slice_sizes=(1,),
                    mode=lax.GatherScatterMode.PROMISE_IN_BOUNDS)


def _rank_last(b):
  lane = lax.iota(jnp.int32, 16)
  rank = jnp.zeros((16,), jnp.int32)
  later = jnp.zeros((16,), jnp.bool_)
  for s in range(1, 16):
    back = _take16(b, jnp.maximum(lane - s, 0))
    fwd = _take16(b, jnp.minimum(lane + s, 15))
    eqb = jnp.logical_and(lane >= s, back == b)
    eqf = jnp.logical_and(lane < 16 - s, fwd == b)
    rank = rank + jnp.where(eqb, 1, 0).astype(jnp.int32)
    later = jnp.logical_or(later, eqf)
  return rank, jnp.logical_not(later)


def _count_body(dst_hbm, cnt_out, cnt_v, dst_v):
  core = lax.axis_index("c")
  sub = lax.axis_index("s")
  wid = core * NS + sub
  zero16 = jnp.zeros((16,), jnp.int32)

  def zc(i, _):
    cnt_v[pl.ds(i * 16, 16)] = zero16
    return 0
  lax.fori_loop(0, NPAD // 16, zc, 0)

  base = wid * EPW

  def blk(j, _):
    off = pl.multiple_of(base + j * K1_BLK, 8)
    pltpu.sync_copy(dst_hbm.at[pl.ds(off, K1_BLK)], dst_v)

    def vec(v, _):
      d = dst_v[pl.ds(v * 16, 16)]
      rank, last = _rank_last(d)
      cur = plsc.load_gather(cnt_v, [d])
      plsc.store_scatter(cnt_v, [d], cur + rank + 1, mask=last)
      return 0
    lax.fori_loop(0, K1_BLK // 16, vec, 0)
    return 0
  lax.fori_loop(0, EPW // K1_BLK, blk, 0)

  pltpu.sync_copy(cnt_v, cnt_out.at[wid])


_count_call = pl.kernel(
    _count_body,
    out_type=jax.ShapeDtypeStruct((NW, NPAD), jnp.int32),
    mesh=_mesh,
    compiler_params=_sc_params,
    scratch_types=(
        pltpu.VMEM((NPAD,), jnp.int32),
        pltpu.VMEM((K1_BLK,), jnp.int32),
    ),
)


def _csr_body(src_hbm, dst_hbm, cnt_hbm, rec_out, rs_out,
              myoff_v, cblk_v, src_v, dst_v, orec_v, oidx_v, tail_v, sem):
  core = lax.axis_index("c")
  sub = lax.axis_index("s")
  wid = core * NS + sub
  lane = lax.iota(jnp.int32, 16)

  def col_blk(cb, carry):
    pltpu.sync_copy(cnt_hbm.at[:, pl.ds(cb * CNT_COL, CNT_COL)], cblk_v)

    def col_vec(g, cy):
      def rowacc(t, a):
        return a + cblk_v[t, pl.ds(g * 16, 16)]
      tot = lax.fori_loop(0, NW, rowacc, jnp.zeros((16,), jnp.int32))
      pref = lax.fori_loop(0, wid, rowacc, jnp.zeros((16,), jnp.int32))
      incl = plsc.cumsum(tot)
      myoff_v[pl.ds(cb * CNT_COL + g * 16, 16)] = incl - tot + cy + pref
      return cy + jnp.sum(tot)
    return lax.fori_loop(0, CNT_COL // 16, col_vec, carry)
  total = lax.fori_loop(0, NPAD // CNT_COL, col_blk, jnp.int32(0))

  @pl.when(wid == 0)
  def _():
    pltpu.sync_copy(myoff_v, rs_out.at[pl.ds(0, NPAD)])
    tail_v[...] = jnp.broadcast_to(total, (16,))
    pltpu.sync_copy(tail_v, rs_out.at[pl.ds(NPAD, 16)])

  base = wid * EPW
  nblk = (EPW + K2_BLK - 1) // K2_BLK

  def blk(j, _):
    off = pl.multiple_of(base + j * K2_BLK, 8)
    pltpu.sync_copy(src_hbm.at[pl.ds(off, K2_BLK)], src_v)
    pltpu.sync_copy(dst_hbm.at[pl.ds(off, K2_BLK)], dst_v)

    def chunk128(k, _):
      def vec8(w, _):
        v = k * 8 + w
        s = src_v[pl.ds(v * 16, 16)]
        d = dst_v[pl.ds(v * 16, 16)]
        eidx = j * K2_BLK + v * 16 + lane
        ok = eidx < EPW
        ds_ = jnp.where(ok, d, NPAD - 1)
        rank, last = _rank_last(ds_)
        cur = plsc.load_gather(myoff_v, [ds_])
        plsc.store_scatter(myoff_v, [ds_], cur + rank + 1,
                           mask=jnp.logical_and(last, ok))
        pos = jnp.where(ok, cur + rank, E + lane)
        rec = (s << LOC_BITS) | (d & LOC_MASK)
        orec_v[pl.ds(v * 16, 16)] = rec
        oidx_v[k, pl.ds(w * 16, 16)] = pos
        return 0
      lax.fori_loop(0, 8, vec8, 0)
      pltpu.async_copy(orec_v.at[pl.ds(k * 128, 128)],
                       rec_out.at[plsc.Indices(oidx_v.at[k])], sem).wait()
      return 0
    lax.fori_loop(0, K2_BLK // 128, chunk128, 0)
    return 0
  lax.fori_loop(0, nblk, blk, 0)


_csr_call = pl.kernel(
    _csr_body,
    out_type=(
        jax.ShapeDtypeStruct((EP,), jnp.int32),
        jax.ShapeDtypeStruct((NPAD + 16,), jnp.int32),
    ),
    mesh=_mesh,
    compiler_params=_sc_params,
    scratch_types=(
        pltpu.VMEM((NPAD,), jnp.int32),
        pltpu.VMEM((NW, CNT_COL), jnp.int32),
        pltpu.VMEM((K2_BLK,), jnp.int32),
        pltpu.VMEM((K2_BLK,), jnp.int32),
        pltpu.VMEM((K2_BLK,), jnp.int32),
        pltpu.VMEM((K2_BLK // 128, 128), jnp.int32),
        pltpu.VMEM((16,), jnp.int32),
        pltpu.SemaphoreType.DMA,
    ),
)


def _agg_body(rec_hbm, rs_hbm, z_hbm, u_hbm, accum, rsw, recbuf,
              sidx, iloc, rows, gsem, *, d, cr, kb, log_kb):
  nchunks = NPAD // cr
  rounds = -(-nchunks // NW)
  core = lax.axis_index("c")
  sub = lax.axis_index("s")
  wid = core * NS + sub
  lane = lax.iota(jnp.int32, 16)

  def chunk(i, _):
    c = jnp.minimum(wid + i * NW, nchunks - 1)
    c0 = pl.multiple_of(c * cr, 8)
    pltpu.sync_copy(z_hbm.at[pl.ds(c0, cr)], accum)
    pltpu.sync_copy(rs_hbm.at[pl.ds(c0, 16)], rsw)
    r0 = jnp.sum(jnp.where(lane == 0, rsw[...], 0))
    pltpu.sync_copy(rs_hbm.at[pl.ds(c0 + cr, 16)], rsw)
    r1 = jnp.sum(jnp.where(lane == 0, rsw[...], 0))
    a0 = r0 - (r0 & 7)
    nblk = lax.shift_right_arithmetic(r1 - a0 + (kb - 1), log_kb)
    c0lo = c0 & LOC_MASK

    def blk(k, _):
      bstart = pl.multiple_of(a0 + k * kb, 8)
      pltpu.sync_copy(rec_hbm.at[pl.ds(bstart, kb)], recbuf)
      for v in range(kb // 16):
        rec = recbuf[pl.ds(v * 16, 16)]
        slot = bstart + v * 16 + lane
        ok = jnp.logical_and(slot >= r0, slot < r1)
        pad_src = 100096 + (slot & 127)
        rec = jnp.where(ok, rec, pad_src << LOC_BITS)
        s_idx = lax.shift_right_logical(rec, LOC_BITS)
        tmp = (rec & LOC_MASK) - c0lo
        loc = tmp + jnp.where(tmp < 0, LOC_MASK + 1, 0)
        loc = jnp.where(ok, loc, 0)
        sidx[pl.ds(v * 16, 16)] = s_idx
        iloc[pl.ds(v * 16, 16)] = loc
      pltpu.async_copy(z_hbm.at[sidx], rows, gsem).wait()

      def grp(g, _):
        loc16 = iloc[pl.ds(g * 16, 16)]
        for e16 in range(0, 16, 2):
          la = _take16(loc16, jnp.full((16,), e16, jnp.int32))
          lb = _take16(loc16, jnp.full((16,), e16 + 1, jnp.int32))
          ea = jnp.broadcast_to(g * 16 + e16, (16,))
          eb = jnp.broadcast_to(g * 16 + e16 + 1, (16,))
          for s in range(d // 16):
            col = s * 16 + lane
            va = plsc.load_gather(rows, [ea, col])
            vb = plsc.load_gather(rows, [eb, col])
            plsc.addupdate_scatter(accum, [la, col], va)
            plsc.addupdate_scatter(accum, [lb, col], vb)
        return 0
      lax.fori_loop(0, kb // 16, grp, 0)
      return 0
    lax.fori_loop(0, nblk, blk, 0)
    pltpu.sync_copy(accum, u_hbm.at[pl.ds(c0, cr)])
    return 0
  lax.fori_loop(0, rounds, chunk, 0)


def _make_agg(d, cr, kb, log_kb):
  body = functools.partial(_agg_body, d=d, cr=cr, kb=kb, log_kb=log_kb)
  return pl.kernel(
      body,
      out_type=jax.ShapeDtypeStruct((NPAD, d), jnp.float32),
      mesh=_mesh,
      compiler_params=_sc_params,
      scratch_types=(
          pltpu.VMEM((cr, d), jnp.float32),
          pltpu.VMEM((16,), jnp.int32),
          pltpu.VMEM((kb,), jnp.int32),
          pltpu.VMEM((kb,), jnp.int32),
          pltpu.VMEM((kb,), jnp.int32),
          pltpu.VMEM((kb, d), jnp.float32),
          pltpu.SemaphoreType.DMA,
      ),
  )


_agg128 = _make_agg(128, 512, 128, 7)
_agg512 = _make_agg(512, 128, 64, 6)
_agg2048 = _make_agg(2048, 16, 32, 5)


TM = 256


def _dinv_body(cnt_ref, x_ref, dinv_ref, z_ref):
  i = pl.program_id(0)
  deg = jnp.sum(cnt_ref[...], axis=0).astype(jnp.float32) + 1.0
  rows = i * TM + lax.broadcasted_iota(jnp.int32, (TM,), 0)
  dv = jnp.where(rows < N, lax.rsqrt(deg), 0.0)
  dinv_ref[...] = dv[:, None]
  z_ref[...] = dv[:, None] * x_ref[...]


def _dinv_call(cnt, x_p):
  return pl.pallas_call(
      _dinv_body,
      grid=(NPAD // TM,),
      in_specs=[
          pl.BlockSpec((NW, TM), lambda i: (0, i)),
          pl.BlockSpec((TM, D_IN), lambda i: (i, 0)),
      ],
      out_specs=(
          pl.BlockSpec((TM, 1), lambda i: (i, 0)),
          pl.BlockSpec((TM, D_IN), lambda i: (i, 0)),
      ),
      out_shape=(
          jax.ShapeDtypeStruct((NPAD, 1), jnp.float32),
          jax.ShapeDtypeStruct((NPAD, D_IN), jnp.float32),
      ),
  )(cnt, x_p)


def _mm_body(u_ref, w_ref, b_ref, dinv_ref, o_ref, *, act):
  acc = jnp.dot(u_ref[...], w_ref[...], preferred_element_type=jnp.float32)
  dv = dinv_ref[...]
  y = acc * dv + b_ref[...]
  if act:
    y = jax.nn.silu(y) * dv
  o_ref[...] = y


def _mm_call(u, w, b, dinv, *, act, tn, nrows=NPAD):
  k = u.shape[1]
  dout = w.shape[1]
  nn = dout // tn
  body = functools.partial(_mm_body, act=act)
  return pl.pallas_call(
      body,
      grid=(nn, -(-nrows // TM)),
      in_specs=[
          pl.BlockSpec((TM, k), lambda j, i: (i, 0)),
          pl.BlockSpec((k, tn), lambda j, i: (0, j)),
          pl.BlockSpec((1, tn), lambda j, i: (0, j)),
          pl.BlockSpec((TM, 1), lambda j, i: (i, 0)),
      ],
      out_specs=pl.BlockSpec((TM, tn), lambda j, i: (i, j)),
      out_shape=jax.ShapeDtypeStruct((nrows, dout), jnp.float32),
  )(u, w, b, dinv)


def kernel(hidden_states, edge_index, W1, b1, W2, b2, W3, b3):
  ei = edge_index.astype(jnp.int32)
  src = jnp.pad(ei[0], (0, K2_BLK))
  dst = jnp.pad(ei[1], (0, K2_BLK))
  x_p = jnp.pad(hidden_states, ((0, NPAD - N), (0, 0)))

  cnt = _count_call(dst)
  rec, rs = _csr_call(src, dst, cnt)
  dinv, z1 = _dinv_call(cnt, x_p)

  u1 = _agg128(rec, rs, z1)
  z2 = _mm_call(u1, W1, b1.reshape(1, -1), dinv, act=True, tn=512)
  u2 = _agg512(rec, rs, z2)
  z3 = _mm_call(u2, W2, b2.reshape(1, -1), dinv, act=True, tn=2048)
  u3 = _agg2048(rec, rs, z3)
  return _mm_call(u3, W3, b3.reshape(1, -1), dinv, act=False, tn=2000,
                  nrows=N)


# speedup vs baseline: 1.0918x; 1.0918x over previous
import functools

import jax
import jax.numpy as jnp
from jax import lax
from jax.experimental import pallas as pl
from jax.experimental.pallas import tpu as pltpu
from jax.experimental.pallas import tpu_sc as plsc

N = 100000
E = 1600000
D_IN = 128
D_H = 512
D_FF = 2048
D_OUT = 2000

NC, NS = 2, 16
NW = NC * NS
EPW = E // NW

NPAD = 100352
LOC_BITS = 13
LOC_MASK = (1 << LOC_BITS) - 1
EP = E + 256

K1_BLK = 2000
K2_BLK = 2048
CNT_COL = 512

_mesh = plsc.VectorSubcoreMesh(core_axis_name="c", subcore_axis_name="s")
_sc_params = pltpu.CompilerParams(needs_layout_passes=False)


_GDN = lax.GatherDimensionNumbers(
    offset_dims=(), collapsed_slice_dims=(0,), start_index_map=(0,))


def _take16(v, idx):
  return lax.gather(v, idx[:, None], _GDN, slice_sizes=(1,),
                    mode=lax.GatherScatterMode.PROMISE_IN_BOUNDS)


def _rank_last(b):
  lane = lax.iota(jnp.int32, 16)
  rank = jnp.zeros((16,), jnp.int32)
  later = jnp.zeros((16,), jnp.bool_)
  for s in range(1, 16):
    back = _take16(b, jnp.maximum(lane - s, 0))
    fwd = _take16(b, jnp.minimum(lane + s, 15))
    eqb = jnp.logical_and(lane >= s, back == b)
    eqf = jnp.logical_and(lane < 16 - s, fwd == b)
    rank = rank + jnp.where(eqb, 1, 0).astype(jnp.int32)
    later = jnp.logical_or(later, eqf)
  return rank, jnp.logical_not(later)


def _count_body(dst_hbm, cnt_out, cnt_v, dst_v):
  core = lax.axis_index("c")
  sub = lax.axis_index("s")
  wid = core * NS + sub
  zero16 = jnp.zeros((16,), jnp.int32)

  def zc(i, _):
    cnt_v[pl.ds(i * 16, 16)] = zero16
    return 0
  lax.fori_loop(0, NPAD // 16, zc, 0)

  base = wid * EPW

  def blk(j, _):
    off = pl.multiple_of(base + j * K1_BLK, 8)
    pltpu.sync_copy(dst_hbm.at[pl.ds(off, K1_BLK)], dst_v)

    def vec(v, _):
      d = dst_v[pl.ds(v * 16, 16)]
      rank, last = _rank_last(d)
      cur = plsc.load_gather(cnt_v, [d])
      plsc.store_scatter(cnt_v, [d], cur + rank + 1, mask=last)
      return 0
    lax.fori_loop(0, K1_BLK // 16, vec, 0)
    return 0
  lax.fori_loop(0, EPW // K1_BLK, blk, 0)

  pltpu.sync_copy(cnt_v, cnt_out.at[wid])


_count_call = pl.kernel(
    _count_body,
    out_type=jax.ShapeDtypeStruct((NW, NPAD), jnp.int32),
    mesh=_mesh,
    compiler_params=_sc_params,
    scratch_types=(
        pltpu.VMEM((NPAD,), jnp.int32),
        pltpu.VMEM((K1_BLK,), jnp.int32),
    ),
)


def _csr_body(src_hbm, dst_hbm, cnt_hbm, rec_out, rs_out,
              myoff_v, cblk_v, src_v, dst_v, orec_v, oidx_v, tail_v, sem):
  core = lax.axis_index("c")
  sub = lax.axis_index("s")
  wid = core * NS + sub
  lane = lax.iota(jnp.int32, 16)

  def col_blk(cb, carry):
    pltpu.sync_copy(cnt_hbm.at[:, pl.ds(cb * CNT_COL, CNT_COL)], cblk_v)

    def col_vec(g, cy):
      def rowacc(t, a):
        return a + cblk_v[t, pl.ds(g * 16, 16)]
      tot = lax.fori_loop(0, NW, rowacc, jnp.zeros((16,), jnp.int32))
      pref = lax.fori_loop(0, wid, rowacc, jnp.zeros((16,), jnp.int32))
      incl = plsc.cumsum(tot)
      myoff_v[pl.ds(cb * CNT_COL + g * 16, 16)] = incl - tot + cy + pref
      return cy + jnp.sum(tot)
    return lax.fori_loop(0, CNT_COL // 16, col_vec, carry)
  total = lax.fori_loop(0, NPAD // CNT_COL, col_blk, jnp.int32(0))

  @pl.when(wid == 0)
  def _():
    pltpu.sync_copy(myoff_v, rs_out.at[pl.ds(0, NPAD)])
    tail_v[...] = jnp.broadcast_to(total, (16,))
    pltpu.sync_copy(tail_v, rs_out.at[pl.ds(NPAD, 16)])

  base = wid * EPW
  nblk = (EPW + K2_BLK - 1) // K2_BLK

  def blk(j, _):
    off = pl.multiple_of(base + j * K2_BLK, 8)
    pltpu.sync_copy(src_hbm.at[pl.ds(off, K2_BLK)], src_v)
    pltpu.sync_copy(dst_hbm.at[pl.ds(off, K2_BLK)], dst_v)

    def chunk128(k, _):
      def vec8(w, _):
        v = k * 8 + w
        s = src_v[pl.ds(v * 16, 16)]
        d = dst_v[pl.ds(v * 16, 16)]
        eidx = j * K2_BLK + v * 16 + lane
        ok = eidx < EPW
        ds_ = jnp.where(ok, d, NPAD - 1)
        rank, last = _rank_last(ds_)
        cur = plsc.load_gather(myoff_v, [ds_])
        plsc.store_scatter(myoff_v, [ds_], cur + rank + 1,
                           mask=jnp.logical_and(last, ok))
        pos = jnp.where(ok, cur + rank, E + lane)
        rec = (s << LOC_BITS) | (d & LOC_MASK)
        orec_v[pl.ds(v * 16, 16)] = rec
        oidx_v[k, pl.ds(w * 16, 16)] = pos
        return 0
      lax.fori_loop(0, 8, vec8, 0)
      pltpu.async_copy(orec_v.at[pl.ds(k * 128, 128)],
                       rec_out.at[plsc.Indices(oidx_v.at[k])], sem).wait()
      return 0
    lax.fori_loop(0, K2_BLK // 128, chunk128, 0)
    return 0
  lax.fori_loop(0, nblk, blk, 0)


_csr_call = pl.kernel(
    _csr_body,
    out_type=(
        jax.ShapeDtypeStruct((EP,), jnp.int32),
        jax.ShapeDtypeStruct((NPAD + 16,), jnp.int32),
    ),
    mesh=_mesh,
    compiler_params=_sc_params,
    scratch_types=(
        pltpu.VMEM((NPAD,), jnp.int32),
        pltpu.VMEM((NW, CNT_COL), jnp.int32),
        pltpu.VMEM((K2_BLK,), jnp.int32),
        pltpu.VMEM((K2_BLK,), jnp.int32),
        pltpu.VMEM((K2_BLK,), jnp.int32),
        pltpu.VMEM((K2_BLK // 128, 128), jnp.int32),
        pltpu.VMEM((16,), jnp.int32),
        pltpu.SemaphoreType.DMA,
    ),
)


def _agg_body(rec_hbm, rs_hbm, z_hbm, u_hbm, accum, rsw, recbuf,
              sidx, iloc, rows, gsem, *, d, cr, kb, log_kb):
  nchunks = NPAD // cr
  rounds = -(-nchunks // NW)
  core = lax.axis_index("c")
  sub = lax.axis_index("s")
  wid = core * NS + sub
  lane = lax.iota(jnp.int32, 16)

  def chunk(i, _):
    c = jnp.minimum(wid + i * NW, nchunks - 1)
    c0 = pl.multiple_of(c * cr, 8)
    pltpu.sync_copy(z_hbm.at[pl.ds(c0, cr)], accum)
    pltpu.sync_copy(rs_hbm.at[pl.ds(c0, 16)], rsw)
    r0 = jnp.sum(jnp.where(lane == 0, rsw[...], 0))
    pltpu.sync_copy(rs_hbm.at[pl.ds(c0 + cr, 16)], rsw)
    r1 = jnp.sum(jnp.where(lane == 0, rsw[...], 0))
    a0 = r0 - (r0 & 7)
    nblk = lax.shift_right_arithmetic(r1 - a0 + (kb - 1), log_kb)
    c0lo = c0 & LOC_MASK

    def blk(k, _):
      bstart = pl.multiple_of(a0 + k * kb, 8)
      pltpu.sync_copy(rec_hbm.at[pl.ds(bstart, kb)], recbuf)
      for v in range(kb // 16):
        rec = recbuf[pl.ds(v * 16, 16)]
        slot = bstart + v * 16 + lane
        ok = jnp.logical_and(slot >= r0, slot < r1)
        pad_src = 100096 + (slot & 127)
        rec = jnp.where(ok, rec, pad_src << LOC_BITS)
        s_idx = lax.shift_right_logical(rec, LOC_BITS)
        tmp = (rec & LOC_MASK) - c0lo
        loc = tmp + jnp.where(tmp < 0, LOC_MASK + 1, 0)
        loc = jnp.where(ok, loc, 0)
        sidx[pl.ds(v * 16, 16)] = s_idx
        iloc[pl.ds(v * 16, 16)] = loc
      pltpu.async_copy(z_hbm.at[sidx], rows, gsem).wait()

      def grp(g, _):
        loc16 = iloc[pl.ds(g * 16, 16)]
        for e16 in range(16):
          l_b = jnp.broadcast_to(
              jnp.sum(jnp.where(lane == e16, loc16, 0)), (16,))
          e_b = jnp.broadcast_to(g * 16 + e16, (16,))
          for s in range(d // 16):
            col = s * 16 + lane
            vals = plsc.load_gather(rows, [e_b, col])
            plsc.addupdate_scatter(accum, [l_b, col], vals)
        return 0
      lax.fori_loop(0, kb // 16, grp, 0)
      return 0
    lax.fori_loop(0, nblk, blk, 0)
    pltpu.sync_copy(accum, u_hbm.at[pl.ds(c0, cr)])
    return 0
  lax.fori_loop(0, rounds, chunk, 0)


def _make_agg(d, cr, kb, log_kb):
  body = functools.partial(_agg_body, d=d, cr=cr, kb=kb, log_kb=log_kb)
  return pl.kernel(
      body,
      out_type=jax.ShapeDtypeStruct((NPAD, d), jnp.float32),
      mesh=_mesh,
      compiler_params=_sc_params,
      scratch_types=(
          pltpu.VMEM((cr, d), jnp.float32),
          pltpu.VMEM((16,), jnp.int32),
          pltpu.VMEM((kb,), jnp.int32),
          pltpu.VMEM((kb,), jnp.int32),
          pltpu.VMEM((kb,), jnp.int32),
          pltpu.VMEM((kb, d), jnp.float32),
          pltpu.SemaphoreType.DMA,
      ),
  )


_agg128 = _make_agg(128, 512, 128, 7)
_agg512 = _make_agg(512, 128, 64, 6)
_agg2048 = _make_agg(2048, 32, 16, 4)


TM = 256


def _dinv_body(cnt_ref, x_ref, dinv_ref, z_ref):
  i = pl.program_id(0)
  deg = jnp.sum(cnt_ref[...], axis=0).astype(jnp.float32) + 1.0
  rows = i * TM + lax.broadcasted_iota(jnp.int32, (TM,), 0)
  dv = jnp.where(rows < N, lax.rsqrt(deg), 0.0)
  dinv_ref[...] = dv[:, None]
  z_ref[...] = dv[:, None] * x_ref[...]


def _dinv_call(cnt, x_p):
  return pl.pallas_call(
      _dinv_body,
      grid=(NPAD // TM,),
      in_specs=[
          pl.BlockSpec((NW, TM), lambda i: (0, i)),
          pl.BlockSpec((TM, D_IN), lambda i: (i, 0)),
      ],
      out_specs=(
          pl.BlockSpec((TM, 1), lambda i: (i, 0)),
          pl.BlockSpec((TM, D_IN), lambda i: (i, 0)),
      ),
      out_shape=(
          jax.ShapeDtypeStruct((NPAD, 1), jnp.float32),
          jax.ShapeDtypeStruct((NPAD, D_IN), jnp.float32),
      ),
  )(cnt, x_p)


def _mm_body(u_ref, w_ref, b_ref, dinv_ref, o_ref, *, act):
  acc = jnp.dot(u_ref[...], w_ref[...], preferred_element_type=jnp.float32)
  dv = dinv_ref[...]
  y = acc * dv + b_ref[...]
  if act:
    y = jax.nn.silu(y) * dv
  o_ref[...] = y


def _mm_call(u, w, b, dinv, *, act, tn, nrows=NPAD):
  k = u.shape[1]
  dout = w.shape[1]
  nn = dout // tn
  body = functools.partial(_mm_body, act=act)
  return pl.pallas_call(
      body,
      grid=(nn, -(-nrows // TM)),
      in_specs=[
          pl.BlockSpec((TM, k), lambda j, i: (i, 0)),
          pl.BlockSpec((k, tn), lambda j, i: (0, j)),
          pl.BlockSpec((1, tn), lambda j, i: (0, j)),
          pl.BlockSpec((TM, 1), lambda j, i: (i, 0)),
      ],
      out_specs=pl.BlockSpec((TM, tn), lambda j, i: (i, j)),
      out_shape=jax.ShapeDtypeStruct((nrows, dout), jnp.float32),
  )(u, w, b, dinv)


def kernel(hidden_states, edge_index, W1, b1, W2, b2, W3, b3):
  ei = edge_index.astype(jnp.int32)
  src = jnp.pad(ei[0], (0, K2_BLK))
  dst = jnp.pad(ei[1], (0, K2_BLK))
  x_p = jnp.pad(hidden_states, ((0, NPAD - N), (0, 0)))

  cnt = _count_call(dst)
  rec, rs = _csr_call(src, dst, cnt)
  dinv, z1 = _dinv_call(cnt, x_p)

  u1 = _agg128(rec, rs, z1)
  z2 = _mm_call(u1, W1, b1.reshape(1, -1), dinv, act=True, tn=512)
  u2 = _agg512(rec, rs, z2)
  z3 = _mm_call(u2, W2, b2.reshape(1, -1), dinv, act=True, tn=2048)
  u3 = _agg2048(rec, rs, z3)
  return _mm_call(u3, W3, b3.reshape(1, -1), dinv, act=False, tn=2000,
                  nrows=N)
